# Initial kernel scaffold; baseline (speedup 1.0000x reference)
import jax
import jax.numpy as jnp
from jax.experimental import pallas as pl


def kernel(x_room, x_device, x_property, x_outside, x_time, ei_room_h, ei_room_v, ei_dev_room, ei_prop_dev, ei_out_room, ei_time_room, ei_time_dev, ei_time_prop, Wt1_property, bt1_property, Wt2_property, bt2_property, Wt1_outside, bt1_outside, Wt2_outside, bt2_outside, Wt1_time, bt1_time, Wt2_time, bt2_time, Wg_h, bg_h, Wg_v, bg_v, Wl_dr, bl_dr, Wr_dr, Wl_pd, bl_pd, Wr_pd, Wl_or, bl_or, Wr_or, Wl_tr, bl_tr, Wr_tr, Wl_td, bl_td, Wr_td, Wl_tp, bl_tp, Wr_tp, ln_g_room, ln_b_room, ln_g_device, ln_b_device, ln_g_property, ln_b_property, ln_g_outside, ln_b_outside, ln_g_time, ln_b_time):
    raise NotImplementedError("write your pallas kernel here")


# speedup vs baseline: 1.9908x; 1.9908x over previous
import functools

import jax
import jax.numpy as jnp
from jax.experimental import pallas as pl

_KT = 3
_C = 128


def _pad_rows(x, block):
    m = x.shape[0]
    mp = (m + block - 1) // block * block
    if mp == m:
        return x, m
    return jnp.pad(x, ((0, mp - m),) + ((0, 0),) * (x.ndim - 1)), m


def _mm_kernel(x_ref, w_ref, b_ref, o_ref):
    y = jnp.dot(x_ref[...], w_ref[...], preferred_element_type=jnp.float32)
    o_ref[...] = jnp.maximum(y + b_ref[...], 0.0)


def _combine_call(x, w, b, block=512):
    xp, m = _pad_rows(x, block)
    mp, k = xp.shape
    n = w.shape[1]
    out = pl.pallas_call(
        _mm_kernel,
        grid=(mp // block,),
        in_specs=[
            pl.BlockSpec((block, k), lambda i: (i, 0)),
            pl.BlockSpec((k, n), lambda i: (0, 0)),
            pl.BlockSpec((1, n), lambda i: (0, 0)),
        ],
        out_specs=pl.BlockSpec((block, n), lambda i: (i, 0)),
        out_shape=jax.ShapeDtypeStruct((mp, n), jnp.float32),
    )(xp, w, b[None])
    return out[:m]


def _tconv_kernel(x_ref, w_ref, b_ref, o_ref, *, tout, relu_in):
    x = x_ref[...]
    if relu_in:
        x = jnp.maximum(x, 0.0)
    b = b_ref[0]
    for t in range(tout):
        y = (jnp.dot(x[:, t], w_ref[0], preferred_element_type=jnp.float32)
             + jnp.dot(x[:, t + 1], w_ref[1], preferred_element_type=jnp.float32)
             + jnp.dot(x[:, t + 2], w_ref[2], preferred_element_type=jnp.float32)
             + b)
        o_ref[:, t, :] = (y[:, :_C] + x[:, t + 2]) * jax.nn.sigmoid(y[:, _C:])


def _tconv_call(x, w, b, relu_in, block=256):
    tin = x.shape[1]
    tout = tin - _KT + 1
    wk = jnp.transpose(w[:, :, :, 0], (2, 1, 0))
    xp, m = _pad_rows(x, block)
    mp = xp.shape[0]
    out = pl.pallas_call(
        functools.partial(_tconv_kernel, tout=tout, relu_in=relu_in),
        grid=(mp // block,),
        in_specs=[
            pl.BlockSpec((block, tin, _C), lambda i: (i, 0, 0)),
            pl.BlockSpec((_KT, _C, 2 * _C), lambda i: (0, 0, 0)),
            pl.BlockSpec((1, 2 * _C), lambda i: (0, 0)),
        ],
        out_specs=pl.BlockSpec((block, tout, _C), lambda i: (i, 0, 0)),
        out_shape=jax.ShapeDtypeStruct((mp, tout, _C), jnp.float32),
    )(xp, wk, b[None])
    return out[:m]


def _ln_kernel(x_ref, g_ref, b_ref, o_ref):
    x = x_ref[...]
    mu = jnp.mean(x, axis=-1, keepdims=True)
    var = jnp.mean((x - mu) ** 2, axis=-1, keepdims=True)
    o_ref[...] = (x - mu) / jnp.sqrt(var + 1e-5) * g_ref[...] + b_ref[...]


def _ln_call(x, g, b, block=512):
    xp, m = _pad_rows(x, block)
    mp = xp.shape[0]
    out = pl.pallas_call(
        _ln_kernel,
        grid=(mp // block,),
        in_specs=[
            pl.BlockSpec((block, _C), lambda i: (i, 0)),
            pl.BlockSpec((1, _C), lambda i: (0, 0)),
            pl.BlockSpec((1, _C), lambda i: (0, 0)),
        ],
        out_specs=pl.BlockSpec((block, _C), lambda i: (i, 0)),
        out_shape=jax.ShapeDtypeStruct((mp, _C), jnp.float32),
    )(xp, g[None], b[None])
    return out[:m]


def _gcn_agg(x, ei, m):
    ones = jnp.ones((ei.shape[1],), jnp.float32)
    deg = jax.ops.segment_sum(ones, ei[1], num_segments=m) + 1.0
    dinv = 1.0 / jnp.sqrt(deg)
    t = x * dinv[:, None]
    s = jax.ops.segment_sum(t[ei[0]], ei[1], num_segments=m)
    return (s + t) * dinv[:, None]


def _sage_mean(xs, ei, md):
    ones = jnp.ones((ei.shape[1],), jnp.float32)
    cnt = jax.ops.segment_sum(ones, ei[1], num_segments=md)
    s = jax.ops.segment_sum(xs[ei[0]], ei[1], num_segments=md)
    return s / jnp.maximum(cnt, 1.0)[:, None]


def kernel(x_room, x_device, x_property, x_outside, x_time, ei_room_h, ei_room_v, ei_dev_room, ei_prop_dev, ei_out_room, ei_time_room, ei_time_dev, ei_time_prop, Wt1_property, bt1_property, Wt2_property, bt2_property, Wt1_outside, bt1_outside, Wt2_outside, bt2_outside, Wt1_time, bt1_time, Wt2_time, bt2_time, Wg_h, bg_h, Wg_v, bg_v, Wl_dr, bl_dr, Wr_dr, Wl_pd, bl_pd, Wr_pd, Wl_or, bl_or, Wr_or, Wl_tr, bl_tr, Wr_tr, Wl_td, bl_td, Wr_td, Wl_tp, bl_tp, Wr_tp, ln_g_room, ln_b_room, ln_g_device, ln_b_device, ln_g_property, ln_b_property, ln_g_outside, ln_b_outside, ln_g_time, ln_b_time):
    B, _, T, _ = x_room.shape
    t1 = T - _KT + 1
    t2 = t1 - _KT + 1

    def to_rows(x):
        return jnp.transpose(x, (0, 3, 2, 1)).reshape(-1, x.shape[2], _C)

    def from_rows(x, n):
        return jnp.transpose(x.reshape(B, n, x.shape[1], _C), (0, 3, 2, 1))

    nv = {"room": x_room.shape[3], "device": x_device.shape[3],
          "property": x_property.shape[3], "outside": x_outside.shape[3],
          "time": x_time.shape[3]}
    m = {k: B * v for k, v in nv.items()}

    f = {}
    f["room"] = to_rows(x_room)[:, :t1]
    f["device"] = to_rows(x_device)[:, :t1]
    f["property"] = _tconv_call(to_rows(x_property), Wt1_property, bt1_property, False)
    f["outside"] = _tconv_call(to_rows(x_outside), Wt1_outside, bt1_outside, False)
    f["time"] = _tconv_call(to_rows(x_time), Wt1_time, bt1_time, False)

    flat = {k: v.reshape(m[k], t1 * _C) for k, v in f.items()}

    agg_h = _gcn_agg(flat["room"], ei_room_h, m["room"])
    agg_v = _gcn_agg(flat["room"], ei_room_v, m["room"])
    mean_dr = _sage_mean(flat["device"], ei_dev_room, m["room"])
    mean_or = _sage_mean(flat["outside"], ei_out_room, m["room"])
    mean_tr = _sage_mean(flat["time"], ei_time_room, m["room"])
    mean_pd = _sage_mean(flat["property"], ei_prop_dev, m["device"])
    mean_td = _sage_mean(flat["time"], ei_time_dev, m["device"])
    mean_tp = _sage_mean(flat["time"], ei_time_prop, m["property"])

    def rows(a):
        return a.reshape(-1, _C)

    room_in = jnp.concatenate(
        [rows(agg_h), rows(agg_v), rows(mean_dr), rows(mean_or),
         rows(mean_tr), f["room"].reshape(-1, _C)], axis=1)
    w_room = jnp.concatenate(
        [Wg_h, Wg_v, Wl_dr, Wl_or, Wl_tr, Wr_dr + Wr_or + Wr_tr], axis=0)
    b_room = bg_h + bg_v + bl_dr + bl_or + bl_tr
    sp_room = _combine_call(room_in, w_room, b_room).reshape(m["room"], t1, _C)

    dev_in = jnp.concatenate(
        [rows(mean_pd), rows(mean_td), f["device"].reshape(-1, _C)], axis=1)
    w_dev = jnp.concatenate([Wl_pd, Wl_td, Wr_pd + Wr_td], axis=0)
    b_dev = bl_pd + bl_td
    sp_dev = _combine_call(dev_in, w_dev, b_dev).reshape(m["device"], t1, _C)

    prop_in = jnp.concatenate(
        [rows(mean_tp), f["property"].reshape(-1, _C)], axis=1)
    w_prop = jnp.concatenate([Wl_tp, Wr_tp], axis=0)
    sp_prop = _combine_call(prop_in, w_prop, bl_tp).reshape(m["property"], t1, _C)

    out_room = _ln_call(jnp.maximum(sp_room[:, :t2], 0.0).reshape(-1, _C),
                        ln_g_room, ln_b_room).reshape(m["room"], t2, _C)
    out_dev = _ln_call(jnp.maximum(sp_dev[:, :t2], 0.0).reshape(-1, _C),
                       ln_g_device, ln_b_device).reshape(m["device"], t2, _C)

    o_prop = _tconv_call(jnp.maximum(sp_prop, 0.0), Wt2_property, bt2_property, False)
    out_prop = _ln_call(o_prop.reshape(-1, _C), ln_g_property,
                        ln_b_property).reshape(m["property"], t2, _C)

    o_out = _tconv_call(f["outside"], Wt2_outside, bt2_outside, True)
    out_out = _ln_call(o_out.reshape(-1, _C), ln_g_outside,
                       ln_b_outside).reshape(m["outside"], t2, _C)

    o_time = _tconv_call(f["time"], Wt2_time, bt2_time, True)
    out_time = _ln_call(o_time.reshape(-1, _C), ln_g_time,
                        ln_b_time).reshape(m["time"], t2, _C)

    return (from_rows(out_room, nv["room"]),
            from_rows(out_dev, nv["device"]),
            from_rows(out_prop, nv["property"]),
            from_rows(out_out, nv["outside"]),
            from_rows(out_time, nv["time"]))


# speedup vs baseline: 2.0267x; 1.0180x over previous
import functools

import jax
import jax.numpy as jnp
from jax.experimental import pallas as pl

_KT = 3
_C = 128


def _pad_rows(x, block):
    m = x.shape[0]
    mp = (m + block - 1) // block * block
    if mp == m:
        return x, m
    return jnp.pad(x, ((0, mp - m),) + ((0, 0),) * (x.ndim - 1)), m


def _mm_kernel(*refs, nx):
    b_ref = refs[2 * nx]
    o_ref = refs[2 * nx + 1]
    y = b_ref[...]
    for j in range(nx):
        y = y + jnp.dot(refs[j][...], refs[nx + j][...],
                        preferred_element_type=jnp.float32)
    o_ref[...] = jnp.maximum(y, 0.0)


def _combine_call(xs, ws, b, block=512):
    nx = len(xs)
    padded = [_pad_rows(x, block) for x in xs]
    m = padded[0][1]
    mp = padded[0][0].shape[0]
    n = ws[0].shape[1]
    in_specs = ([pl.BlockSpec((block, x.shape[1]), lambda i: (i, 0))
                 for x, _ in padded]
                + [pl.BlockSpec(w.shape, lambda i: (0, 0)) for w in ws]
                + [pl.BlockSpec((1, n), lambda i: (0, 0))])
    out = pl.pallas_call(
        functools.partial(_mm_kernel, nx=nx),
        grid=(mp // block,),
        in_specs=in_specs,
        out_specs=pl.BlockSpec((block, n), lambda i: (i, 0)),
        out_shape=jax.ShapeDtypeStruct((mp, n), jnp.float32),
    )(*[x for x, _ in padded], *ws, b[None])
    return out[:m]


def _tconv_kernel(x_ref, w_ref, b_ref, o_ref, *, tout, relu_in):
    x = x_ref[...]
    if relu_in:
        x = jnp.maximum(x, 0.0)
    b = b_ref[0]
    for t in range(tout):
        y = (jnp.dot(x[:, t], w_ref[0], preferred_element_type=jnp.float32)
             + jnp.dot(x[:, t + 1], w_ref[1], preferred_element_type=jnp.float32)
             + jnp.dot(x[:, t + 2], w_ref[2], preferred_element_type=jnp.float32)
             + b)
        o_ref[:, t, :] = (y[:, :_C] + x[:, t + 2]) * jax.nn.sigmoid(y[:, _C:])


def _tconv_call(x, w, b, relu_in, block=256):
    tin = x.shape[1]
    tout = tin - _KT + 1
    wk = jnp.transpose(w[:, :, :, 0], (2, 1, 0))
    xp, m = _pad_rows(x, block)
    mp = xp.shape[0]
    out = pl.pallas_call(
        functools.partial(_tconv_kernel, tout=tout, relu_in=relu_in),
        grid=(mp // block,),
        in_specs=[
            pl.BlockSpec((block, tin, _C), lambda i: (i, 0, 0)),
            pl.BlockSpec((_KT, _C, 2 * _C), lambda i: (0, 0, 0)),
            pl.BlockSpec((1, 2 * _C), lambda i: (0, 0)),
        ],
        out_specs=pl.BlockSpec((block, tout, _C), lambda i: (i, 0, 0)),
        out_shape=jax.ShapeDtypeStruct((mp, tout, _C), jnp.float32),
    )(xp, wk, b[None])
    return out[:m]


def _ln_kernel(x_ref, g_ref, b_ref, o_ref, *, relu_in=False):
    x = x_ref[...]
    if relu_in:
        x = jnp.maximum(x, 0.0)
    mu = jnp.mean(x, axis=-1, keepdims=True)
    var = jnp.mean((x - mu) ** 2, axis=-1, keepdims=True)
    o_ref[...] = (x - mu) / jnp.sqrt(var + 1e-5) * g_ref[...] + b_ref[...]


def _ln_call(x, g, b, relu_in=False, block=512):
    xp, m = _pad_rows(x, block)
    mp = xp.shape[0]
    out = pl.pallas_call(
        functools.partial(_ln_kernel, relu_in=relu_in),
        grid=(mp // block,),
        in_specs=[
            pl.BlockSpec((block, _C), lambda i: (i, 0)),
            pl.BlockSpec((1, _C), lambda i: (0, 0)),
            pl.BlockSpec((1, _C), lambda i: (0, 0)),
        ],
        out_specs=pl.BlockSpec((block, _C), lambda i: (i, 0)),
        out_shape=jax.ShapeDtypeStruct((mp, _C), jnp.float32),
    )(xp, g[None], b[None])
    return out[:m]


def _gcn_agg(x, ei, m):
    ones = jnp.ones((ei.shape[1],), jnp.float32)
    deg = jax.ops.segment_sum(ones, ei[1], num_segments=m) + 1.0
    dinv = 1.0 / jnp.sqrt(deg)
    t = x * dinv[:, None]
    s = jax.ops.segment_sum(t[ei[0]], ei[1], num_segments=m)
    return (s + t) * dinv[:, None]


def _sage_mean(xs, ei, md):
    ones = jnp.ones((ei.shape[1],), jnp.float32)
    cnt = jax.ops.segment_sum(ones, ei[1], num_segments=md)
    s = jax.ops.segment_sum(xs[ei[0]], ei[1], num_segments=md)
    return s / jnp.maximum(cnt, 1.0)[:, None]


def kernel(x_room, x_device, x_property, x_outside, x_time, ei_room_h, ei_room_v, ei_dev_room, ei_prop_dev, ei_out_room, ei_time_room, ei_time_dev, ei_time_prop, Wt1_property, bt1_property, Wt2_property, bt2_property, Wt1_outside, bt1_outside, Wt2_outside, bt2_outside, Wt1_time, bt1_time, Wt2_time, bt2_time, Wg_h, bg_h, Wg_v, bg_v, Wl_dr, bl_dr, Wr_dr, Wl_pd, bl_pd, Wr_pd, Wl_or, bl_or, Wr_or, Wl_tr, bl_tr, Wr_tr, Wl_td, bl_td, Wr_td, Wl_tp, bl_tp, Wr_tp, ln_g_room, ln_b_room, ln_g_device, ln_b_device, ln_g_property, ln_b_property, ln_g_outside, ln_b_outside, ln_g_time, ln_b_time):
    B, _, T, _ = x_room.shape
    t1 = T - _KT + 1
    t2 = t1 - _KT + 1

    def to_rows(x):
        return jnp.transpose(x, (0, 3, 2, 1)).reshape(-1, x.shape[2], _C)

    def from_rows(x, n):
        return jnp.transpose(x.reshape(B, n, x.shape[1], _C), (0, 3, 2, 1))

    nv = {"room": x_room.shape[3], "device": x_device.shape[3],
          "property": x_property.shape[3], "outside": x_outside.shape[3],
          "time": x_time.shape[3]}
    m = {k: B * v for k, v in nv.items()}

    f = {}
    f["room"] = to_rows(x_room)[:, :t1]
    f["device"] = to_rows(x_device)[:, :t1]
    f["property"] = _tconv_call(to_rows(x_property), Wt1_property, bt1_property, False)
    f["outside"] = _tconv_call(to_rows(x_outside), Wt1_outside, bt1_outside, False)
    f["time"] = _tconv_call(to_rows(x_time), Wt1_time, bt1_time, False)

    flat = {k: v.reshape(m[k], t1 * _C) for k, v in f.items()}

    agg_h = _gcn_agg(flat["room"], ei_room_h, m["room"])
    agg_v = _gcn_agg(flat["room"], ei_room_v, m["room"])
    mean_dr = _sage_mean(flat["device"], ei_dev_room, m["room"])
    mean_or = _sage_mean(flat["outside"], ei_out_room, m["room"])
    mean_tr = _sage_mean(flat["time"], ei_time_room, m["room"])
    mean_pd = _sage_mean(flat["property"], ei_prop_dev, m["device"])
    mean_td = _sage_mean(flat["time"], ei_time_dev, m["device"])
    mean_tp = _sage_mean(flat["time"], ei_time_prop, m["property"])

    def rows(a):
        return a.reshape(-1, _C)

    sp_room = _combine_call(
        [rows(agg_h), rows(agg_v), rows(mean_dr), rows(mean_or),
         rows(mean_tr), f["room"].reshape(-1, _C)],
        [Wg_h, Wg_v, Wl_dr, Wl_or, Wl_tr, Wr_dr + Wr_or + Wr_tr],
        bg_h + bg_v + bl_dr + bl_or + bl_tr).reshape(m["room"], t1, _C)

    sp_dev = _combine_call(
        [rows(mean_pd), rows(mean_td), f["device"].reshape(-1, _C)],
        [Wl_pd, Wl_td, Wr_pd + Wr_td],
        bl_pd + bl_td).reshape(m["device"], t1, _C)

    sp_prop = _combine_call(
        [rows(mean_tp), f["property"].reshape(-1, _C)],
        [Wl_tp, Wr_tp], bl_tp).reshape(m["property"], t1, _C)

    out_room = _ln_call(sp_room[:, :t2].reshape(-1, _C),
                        ln_g_room, ln_b_room).reshape(m["room"], t2, _C)
    out_dev = _ln_call(sp_dev[:, :t2].reshape(-1, _C),
                       ln_g_device, ln_b_device).reshape(m["device"], t2, _C)

    o_prop = _tconv_call(sp_prop, Wt2_property, bt2_property, False)
    out_prop = _ln_call(o_prop.reshape(-1, _C), ln_g_property,
                        ln_b_property).reshape(m["property"], t2, _C)

    o_out = _tconv_call(f["outside"], Wt2_outside, bt2_outside, True)
    out_out = _ln_call(o_out.reshape(-1, _C), ln_g_outside,
                       ln_b_outside).reshape(m["outside"], t2, _C)

    o_time = _tconv_call(f["time"], Wt2_time, bt2_time, True)
    out_time = _ln_call(o_time.reshape(-1, _C), ln_g_time,
                        ln_b_time).reshape(m["time"], t2, _C)

    return (from_rows(out_room, nv["room"]),
            from_rows(out_dev, nv["device"]),
            from_rows(out_prop, nv["property"]),
            from_rows(out_out, nv["outside"]),
            from_rows(out_time, nv["time"]))
